# baseline (speedup 1.0000x reference)
import functools

import jax
import jax.numpy as jnp
from jax import lax
from jax.experimental import pallas as pl
from jax.experimental.pallas import tpu as pltpu
from jax.experimental.pallas import tpu_sc as plsc

_H = 26
_D = 64
_LANES = 16
_V = 2600000
_NT_REAL = _V // 128
_REM = _V - _NT_REAL * 128
_TPW = 636
_NT_PAD = 32 * _TPW
_R1 = (_NT_REAL + 1 + (_NT_PAD - _NT_REAL) + 7) // 8 * 8


def _transpose_body(tab_t, out1, in_v, out_v, tr_v, rsem, wsem):
  wid = lax.axis_index("s") * 2 + lax.axis_index("c")
  lane = lax.iota(jnp.int32, _LANES)
  dq = [q * _LANES + lane for q in range(4)]

  def tile_id(k):
    return wid + 32 * k

  def read_desc(k, b):
    t = tile_id(k)
    tr = jnp.where(t < _NT_REAL, t, 0)
    return pltpu.make_async_copy(
        tab_t.at[:, pl.ds(tr * 128, 128)], in_v.at[b], rsem.at[b])

  def write_desc(k, b):
    t = tile_id(k)
    tw = jnp.where(t < _NT_REAL, t, t - _NT_REAL + _NT_REAL + 1)
    return pltpu.make_async_copy(out_v.at[b], out1.at[tw], wsem.at[b])

  def transpose(b, ncols):
    def row(r, _):
      rv = jnp.full((_LANES,), r, jnp.int32)
      for q in range(4):
        out_v[b, pl.ds(r * _D + q * _LANES, _LANES)] = plsc.load_gather(
            in_v.at[b], [dq[q], rv])
      return 0
    lax.fori_loop(0, ncols, row, 0, unroll=4)

  def body_one(k, b, fire_next):
    read_desc(k, b).wait()
    transpose(b, 128)
    write_desc(k, b).start()
    if fire_next:
      read_desc(k + 2, b).start()

  read_desc(0, 0).start()
  read_desc(1, 1).start()
  body_one(0, 0, True)
  body_one(1, 1, True)

  def step(s, _):
    for b in range(2):
      k = 2 * s + b
      write_desc(k - 2, b).wait()
      body_one(k, b, True)
    return 0

  n_super = _TPW // 2
  lax.fori_loop(1, n_super - 1, step, 0)

  for b in range(2):
    k = 2 * (n_super - 1) + b
    write_desc(k - 2, b).wait()
    body_one(k, b, False)
  for b in range(2):
    write_desc(2 * (n_super - 1) + b, b).wait()

  @pl.when(wid == 0)
  def _():
    pltpu.sync_copy(tab_t.at[:, pl.ds(_NT_REAL * 128, _REM)], tr_v)
    def row(r, _):
      rv = jnp.full((_LANES,), r, jnp.int32)
      for q in range(4):
        out_v[0, pl.ds(r * _D + q * _LANES, _LANES)] = plsc.load_gather(
            tr_v, [dq[q], rv])
      return 0
    lax.fori_loop(0, _REM, row, 0, unroll=4)
    pltpu.sync_copy(out_v.at[0], out1.at[_NT_REAL])


def _gather_body(chunk, group, n_groups, nbuf,
                 ids_hbm, offs_hbm, table_hbm, out_hbm,
                 idx_v, offs_v, rows_v, gsem, wsem):
  wid = lax.axis_index("s") * 2 + lax.axis_index("c")
  base = wid * chunk

  pltpu.sync_copy(ids_hbm.at[pl.ds(base, chunk)], idx_v)
  pltpu.sync_copy(offs_hbm, offs_v)

  def shift(j, _):
    p = j * _LANES
    off = offs_v[pl.ds(lax.rem(p, 8 * _H), _LANES)]
    idx_v[pl.ds(p, _LANES)] = idx_v[pl.ds(p, _LANES)] + off
    return 0

  lax.fori_loop(0, chunk // _LANES, shift, 0, unroll=8)

  def gather_desc(g, b):
    return pltpu.make_async_copy(
        table_hbm.at[idx_v.at[pl.ds(g * group, group)]], rows_v.at[b],
        gsem.at[b])

  def write_desc(g, b):
    return pltpu.make_async_copy(
        rows_v.at[b], out_hbm.at[pl.ds(base + g * group, group)], wsem.at[b])

  for b in range(nbuf):
    gather_desc(b, b).start()

  def step(s, _):
    g0 = s * nbuf
    for b in range(nbuf):
      gather_desc(g0 + b, b).wait()
      write_desc(g0 + b, b).start()
    for b in range(nbuf):
      write_desc(g0 + b, b).wait()
      gather_desc(g0 + nbuf + b, b).start()
    return 0

  n_super = n_groups // nbuf
  lax.fori_loop(0, n_super - 1, step, 0)

  g0 = (n_super - 1) * nbuf
  for b in range(nbuf):
    gather_desc(g0 + b, b).wait()
    write_desc(g0 + b, b).start()
  for b in range(nbuf):
    write_desc(g0 + b, b).wait()


@jax.jit
def _mhe(hash_ids, table, offsets):
  bh = hash_ids.shape[0] * hash_ids.shape[1]
  chunk = bh // 32
  group = 128
  n_groups = chunk // group
  nbuf = 8

  ids_flat = hash_ids.reshape(bh)
  offs_tiled = jnp.tile(offsets, 8)
  tab_t = table.T

  mesh = plsc.VectorSubcoreMesh(core_axis_name="c", subcore_axis_name="s")

  out1 = pl.kernel(
      _transpose_body,
      out_type=jax.ShapeDtypeStruct((_R1, 128 * _D), jnp.float32),
      mesh=mesh,
      compiler_params=pltpu.CompilerParams(needs_layout_passes=False),
      scratch_types=[
          pltpu.VMEM((2, _D, 128), jnp.float32),
          pltpu.VMEM((2, 128 * _D), jnp.float32),
          pltpu.VMEM((_D, _REM), jnp.float32),
          pltpu.SemaphoreType.DMA((2,)),
          pltpu.SemaphoreType.DMA((2,)),
      ],
  )(tab_t)

  table_lin = out1.reshape(_R1 * 128, _D)

  body = functools.partial(_gather_body, chunk, group, n_groups, nbuf)
  out = pl.kernel(
      body,
      out_type=jax.ShapeDtypeStruct((bh, _D), jnp.float32),
      mesh=mesh,
      compiler_params=pltpu.CompilerParams(use_tc_tiling_on_sc=False),
      scratch_types=[
          pltpu.VMEM((chunk,), jnp.int32),
          pltpu.VMEM((8 * _H,), jnp.int32),
          pltpu.VMEM((nbuf, group, _D), jnp.float32),
          pltpu.SemaphoreType.DMA((nbuf,)),
          pltpu.SemaphoreType.DMA((nbuf,)),
      ],
  )(ids_flat, offs_tiled, table_lin)
  return out.reshape(hash_ids.shape[0], hash_ids.shape[1], _D)


def kernel(hash_ids, table, offsets):
  return _mhe(hash_ids, table, offsets)


# speedup vs baseline: 2.5687x; 2.5687x over previous
import functools

import jax
import jax.numpy as jnp
from jax import lax
from jax.experimental import pallas as pl
from jax.experimental.pallas import tpu as pltpu
from jax.experimental.pallas import tpu_sc as plsc

_H = 26
_D = 64
_LANES = 16


def _body(chunk, group, n_groups, nbuf,
          ids_hbm, offs_hbm, table_hbm, out_hbm,
          idx_v, offs_v, rows_v, gsem, wsem):
  wid = lax.axis_index("s") * 2 + lax.axis_index("c")
  base = wid * chunk

  pltpu.sync_copy(ids_hbm.at[pl.ds(base, chunk)], idx_v)
  pltpu.sync_copy(offs_hbm, offs_v)

  def shift(j, _):
    p = j * _LANES
    off = offs_v[pl.ds(lax.rem(p, 8 * _H), _LANES)]
    idx_v[pl.ds(p, _LANES)] = idx_v[pl.ds(p, _LANES)] + off
    return 0

  lax.fori_loop(0, chunk // _LANES, shift, 0, unroll=8)

  def gather_desc(g, b):
    return pltpu.make_async_copy(
        table_hbm.at[idx_v.at[pl.ds(g * group, group)]], rows_v.at[b],
        gsem.at[b])

  def write_desc(g, b):
    return pltpu.make_async_copy(
        rows_v.at[b], out_hbm.at[pl.ds(base + g * group, group)], wsem.at[b])

  for b in range(nbuf):
    gather_desc(b, b).start()

  def step(s, _):
    g0 = s * nbuf
    for b in range(nbuf):
      gather_desc(g0 + b, b).wait()
      write_desc(g0 + b, b).start()
    for b in range(nbuf):
      write_desc(g0 + b, b).wait()
      gather_desc(g0 + nbuf + b, b).start()
    return 0

  n_super = n_groups // nbuf
  lax.fori_loop(0, n_super - 1, step, 0)

  g0 = (n_super - 1) * nbuf
  for b in range(nbuf):
    gather_desc(g0 + b, b).wait()
    write_desc(g0 + b, b).start()
  for b in range(nbuf):
    write_desc(g0 + b, b).wait()


@jax.jit
def _flatten(table):
  return table.reshape(-1)


@jax.jit
def _mhe(hash_ids, table_flat, offsets):
  bh = hash_ids.shape[0] * hash_ids.shape[1]
  info = plsc.get_sparse_core_info()
  nw = info.num_cores * info.num_subcores
  chunk = bh // nw
  group = 128
  n_groups = chunk // group
  nbuf = 8

  ids_flat = hash_ids.reshape(bh)
  offs_tiled = jnp.tile(offsets, 8)
  table = table_flat.reshape(table_flat.shape[0] // _D, _D)

  mesh = plsc.VectorSubcoreMesh(core_axis_name="c", subcore_axis_name="s")
  body = functools.partial(_body, chunk, group, n_groups, nbuf)
  out = pl.kernel(
      body,
      out_type=jax.ShapeDtypeStruct((bh, _D), jnp.float32),
      mesh=mesh,
      compiler_params=pltpu.CompilerParams(use_tc_tiling_on_sc=False),
      scratch_types=[
          pltpu.VMEM((chunk,), jnp.int32),
          pltpu.VMEM((8 * _H,), jnp.int32),
          pltpu.VMEM((nbuf, group, _D), jnp.float32),
          pltpu.SemaphoreType.DMA((nbuf,)),
          pltpu.SemaphoreType.DMA((nbuf,)),
      ],
  )(ids_flat, offs_tiled, table)
  return out.reshape(hash_ids.shape[0], hash_ids.shape[1], _D)


def kernel(hash_ids, table, offsets):
  return _mhe(hash_ids, _flatten(table), offsets)
